# baseline (speedup 1.0000x reference)
import functools
import jax
import jax.numpy as jnp
from jax.experimental import pallas as pl
from jax.experimental.pallas import tpu as pltpu

N_NODES = 5000
N_EDGES = 10000
D_EDGE = 16
DIM = 64
HID = 128

E_BLK = 512
EP = 10240
N_BLK = 256


def _pad_rows(a, n):
    if a.shape[0] == n:
        return a
    return jnp.pad(a, ((0, n - a.shape[0]),) + ((0, 0),) * (a.ndim - 1))


def _msg_body(ea_ref, xs_ref, w1_ref, b1_ref, w2_ref, b2_ref, out_ref):
    h1 = jnp.maximum(
        jnp.dot(ea_ref[...], w1_ref[...], preferred_element_type=jnp.float32, precision=jax.lax.Precision.HIGHEST)
        + b1_ref[...], 0.0)
    w = jnp.dot(h1, w2_ref[...], preferred_element_type=jnp.float32, precision=jax.lax.Precision.HIGHEST) + b2_ref[...]
    w3 = w.reshape(E_BLK, DIM, DIM)
    out_ref[...] = jnp.sum(w3 * xs_ref[...][:, :, None], axis=1)


def _msg_kernel(ea, xs, w1, b1, w2, b2):
    grid = (EP // E_BLK,)
    return pl.pallas_call(
        _msg_body,
        grid=grid,
        in_specs=[
            pl.BlockSpec((E_BLK, D_EDGE), lambda i: (i, 0)),
            pl.BlockSpec((E_BLK, DIM), lambda i: (i, 0)),
            pl.BlockSpec((D_EDGE, HID), lambda i: (0, 0)),
            pl.BlockSpec((1, HID), lambda i: (0, 0)),
            pl.BlockSpec((HID, DIM * DIM), lambda i: (0, 0)),
            pl.BlockSpec((1, DIM * DIM), lambda i: (0, 0)),
        ],
        out_specs=pl.BlockSpec((E_BLK, DIM), lambda i: (i, 0)),
        out_shape=jax.ShapeDtypeStruct((EP, DIM), jnp.float32),
    )(ea, xs, w1, b1.reshape(1, HID), w2, b2.reshape(1, DIM * DIM))


def _combine_body(aggr_ref, x_ref, root_ref, bias_ref, out_ref):
    out_ref[...] = jnp.maximum(
        aggr_ref[...]
        + jnp.dot(x_ref[...], root_ref[...], preferred_element_type=jnp.float32, precision=jax.lax.Precision.HIGHEST)
        + bias_ref[...], 0.0)


def _combine_kernel(aggr, x, root, bias):
    np_ = aggr.shape[0]
    grid = (np_ // N_BLK,)
    return pl.pallas_call(
        _combine_body,
        grid=grid,
        in_specs=[
            pl.BlockSpec((N_BLK, DIM), lambda i: (i, 0)),
            pl.BlockSpec((N_BLK, DIM), lambda i: (i, 0)),
            pl.BlockSpec((DIM, DIM), lambda i: (0, 0)),
            pl.BlockSpec((1, DIM), lambda i: (0, 0)),
        ],
        out_specs=pl.BlockSpec((N_BLK, DIM), lambda i: (i, 0)),
        out_shape=jax.ShapeDtypeStruct((np_, DIM), jnp.float32),
    )(aggr, x, root, bias.reshape(1, DIM))


def _score_body(aggr_ref, h_ref, wrel_ref, brel_ref, wroot_ref,
                score_ref, scaled_ref):
    s = (jnp.sum(aggr_ref[...] * wrel_ref[...], axis=1, keepdims=True)
         + brel_ref[...]
         + jnp.sum(h_ref[...] * wroot_ref[...], axis=1, keepdims=True))
    score_ref[...] = s
    scaled_ref[...] = h_ref[...] * jnp.tanh(s)


def _score_kernel(aggr, h, wrel, brel, wroot):
    np_ = aggr.shape[0]
    grid = (np_ // N_BLK,)
    return pl.pallas_call(
        _score_body,
        grid=grid,
        in_specs=[
            pl.BlockSpec((N_BLK, DIM), lambda i: (i, 0)),
            pl.BlockSpec((N_BLK, DIM), lambda i: (i, 0)),
            pl.BlockSpec((1, DIM), lambda i: (0, 0)),
            pl.BlockSpec((1, 1), lambda i: (0, 0)),
            pl.BlockSpec((1, DIM), lambda i: (0, 0)),
        ],
        out_specs=[
            pl.BlockSpec((N_BLK, 1), lambda i: (i, 0)),
            pl.BlockSpec((N_BLK, DIM), lambda i: (i, 0)),
        ],
        out_shape=[
            jax.ShapeDtypeStruct((np_, 1), jnp.float32),
            jax.ShapeDtypeStruct((np_, DIM), jnp.float32),
        ],
    )(aggr, h, wrel.reshape(1, DIM), brel.reshape(1, 1), wroot.reshape(1, DIM))


def _final_body(h1_ref, h2_ref, h3_ref, w1_ref, b1_ref, w2_ref, b2_ref,
                w3_ref, b3_ref, out_ref, *, k1, k2, k3):
    x1 = jnp.sum(h1_ref[...], axis=0, keepdims=True) * (1.0 / k1)
    x2 = jnp.sum(h2_ref[...], axis=0, keepdims=True) * (1.0 / k2)
    x3 = jnp.sum(h3_ref[...], axis=0, keepdims=True) * (1.0 / k3)
    z = jnp.concatenate([x1, x2, x3], axis=1)
    z = jnp.maximum(
        jnp.dot(z, w1_ref[...], preferred_element_type=jnp.float32, precision=jax.lax.Precision.HIGHEST)
        + b1_ref[...], 0.0)
    z = jnp.maximum(
        jnp.dot(z, w2_ref[...], preferred_element_type=jnp.float32, precision=jax.lax.Precision.HIGHEST)
        + b2_ref[...], 0.0)
    out_ref[...] = (jnp.dot(z, w3_ref[...], preferred_element_type=jnp.float32, precision=jax.lax.Precision.HIGHEST)
                    + b3_ref[...])


def _final_kernel(h1, h2, h3, k1, k2, k3,
                  lin1_w, lin1_b, lin2_w, lin2_b, lin3_w, lin3_b):
    body = functools.partial(_final_body, k1=k1, k2=k2, k3=k3)
    return pl.pallas_call(
        body,
        out_shape=jax.ShapeDtypeStruct((1, 1), jnp.float32),
    )(h1, h2, h3, lin1_w, lin1_b.reshape(1, DIM),
      lin2_w, lin2_b.reshape(1, DIM // 2), lin3_w, lin3_b.reshape(1, 1))


def kernel(x, edge_index, edge_attr, c1_w1, c1_b1, c1_w2, c1_b2, c1_root, c1_bias, c2_w1, c2_b1, c2_w2, c2_b2, c2_root, c2_bias, c3_w1, c3_b1, c3_w2, c3_b2, c3_root, c3_bias, p1_wrel, p1_brel, p1_wroot, p2_wrel, p2_brel, p2_wroot, p3_wrel, p3_brel, p3_wroot, lin1_w, lin1_b, lin2_w, lin2_b, lin3_w, lin3_b):
    src, dst = edge_index[0], edge_index[1]
    mask = jnp.ones((N_EDGES,), dtype=bool)
    eap = _pad_rows(edge_attr, EP)

    conv = [(c1_w1, c1_b1, c1_w2, c1_b2, c1_root, c1_bias),
            (c2_w1, c2_b1, c2_w2, c2_b2, c2_root, c2_bias),
            (c3_w1, c3_b1, c3_w2, c3_b2, c3_root, c3_bias)]
    pool = [(p1_wrel, p1_brel, p1_wroot),
            (p2_wrel, p2_brel, p2_wroot),
            (p3_wrel, p3_brel, p3_wroot)]

    h = x
    n = N_NODES
    pooled = []
    ks = []
    for layer in range(3):
        w1, b1, w2, b2, root, bias = conv[layer]
        wrel, brel, wroot = pool[layer]
        np_ = ((n + N_BLK - 1) // N_BLK) * N_BLK
        hp = _pad_rows(h, np_)

        xs = jnp.where(mask[:, None], hp[src], 0.0)
        msg = _msg_kernel(eap, _pad_rows(xs, EP), w1, b1, w2, b2)[:N_EDGES]
        aggr = jax.ops.segment_sum(msg, dst, num_segments=np_)
        h = _combine_kernel(aggr, hp, root, bias)

        msk_e = mask[:, None].astype(jnp.float32)
        aggr2 = jax.ops.segment_sum(h[src] * msk_e, dst, num_segments=np_)
        score, scaled = _score_kernel(aggr2, h, wrel, brel, wroot)
        score = score[:n, 0]
        k = -(-n // 2)
        _, perm = jax.lax.top_k(score, k)
        new_h = scaled[perm]
        node_map = jnp.zeros((np_,), dtype=src.dtype).at[perm].set(
            jnp.arange(k, dtype=src.dtype))
        sel = jnp.zeros((np_,), dtype=bool).at[perm].set(True)
        mask = mask & sel[src] & sel[dst]
        src = node_map[src]
        dst = node_map[dst]
        kp = ((k + N_BLK - 1) // N_BLK) * N_BLK
        pooled.append(_pad_rows(new_h, kp))
        ks.append(k)
        h = new_h
        n = k

    z = _final_kernel(pooled[0], pooled[1], pooled[2], ks[0], ks[1], ks[2],
                      lin1_w, lin1_b, lin2_w, lin2_b, lin3_w, lin3_b)
    return z.reshape(-1)


# speedup vs baseline: 1.2250x; 1.2250x over previous
import functools
import jax
import jax.numpy as jnp
from jax.experimental import pallas as pl
from jax.experimental.pallas import tpu as pltpu

N_NODES = 5000
N_EDGES = 10000
D_EDGE = 16
DIM = 64
HID = 128

E_BLK = 512
EP = 10240
NP = 5120
N_BLK = 256
R_BLK = 512

_HIGH = jax.lax.Precision.HIGHEST
_NEG = -1.0e30


def _pad_rows(a, n):
    if a.shape[0] == n:
        return a
    return jnp.pad(a, ((0, n - a.shape[0]),) + ((0, 0),) * (a.ndim - 1))


def _msg_body(ea_ref, xs_ref, w1_ref, b1_ref, w2_ref, b2_ref, out_ref):
    h1 = jnp.maximum(
        jnp.dot(ea_ref[...], w1_ref[...], preferred_element_type=jnp.float32,
                precision=_HIGH) + b1_ref[...], 0.0)
    w = jnp.dot(h1, w2_ref[...], preferred_element_type=jnp.float32,
                precision=_HIGH) + b2_ref[...]
    p = w * jnp.repeat(xs_ref[...], DIM, axis=1)
    half = (DIM * DIM) // 2
    while half >= DIM:
        p = p[:, :half] + p[:, half:]
        half //= 2
    out_ref[...] = p


def _msg_kernel(ea, xs, w1, b1, w2, b2):
    grid = (EP // E_BLK,)
    return pl.pallas_call(
        _msg_body,
        grid=grid,
        in_specs=[
            pl.BlockSpec((E_BLK, D_EDGE), lambda i: (i, 0)),
            pl.BlockSpec((E_BLK, DIM), lambda i: (i, 0)),
            pl.BlockSpec((D_EDGE, HID), lambda i: (0, 0)),
            pl.BlockSpec((1, HID), lambda i: (0, 0)),
            pl.BlockSpec((HID, DIM * DIM), lambda i: (0, 0)),
            pl.BlockSpec((1, DIM * DIM), lambda i: (0, 0)),
        ],
        out_specs=pl.BlockSpec((E_BLK, DIM), lambda i: (i, 0)),
        out_shape=jax.ShapeDtypeStruct((EP, DIM), jnp.float32),
    )(ea, xs, w1, b1.reshape(1, HID), w2, b2.reshape(1, DIM * DIM))


def _combine_body(aggr_ref, x_ref, root_ref, bias_ref, out_ref):
    out_ref[...] = jnp.maximum(
        aggr_ref[...]
        + jnp.dot(x_ref[...], root_ref[...], preferred_element_type=jnp.float32,
                  precision=_HIGH)
        + bias_ref[...], 0.0)


def _combine_kernel(aggr, x, root, bias):
    grid = (NP // N_BLK,)
    return pl.pallas_call(
        _combine_body,
        grid=grid,
        in_specs=[
            pl.BlockSpec((N_BLK, DIM), lambda i: (i, 0)),
            pl.BlockSpec((N_BLK, DIM), lambda i: (i, 0)),
            pl.BlockSpec((DIM, DIM), lambda i: (0, 0)),
            pl.BlockSpec((1, DIM), lambda i: (0, 0)),
        ],
        out_specs=pl.BlockSpec((N_BLK, DIM), lambda i: (i, 0)),
        out_shape=jax.ShapeDtypeStruct((NP, DIM), jnp.float32),
    )(aggr, x, root, bias.reshape(1, DIM))


def _score_body(aggr_ref, h_ref, wrel_ref, brel_ref, wroot_ref,
                score_ref, scaled_ref):
    s = (jnp.sum(aggr_ref[...] * wrel_ref[...], axis=1, keepdims=True)
         + brel_ref[...]
         + jnp.sum(h_ref[...] * wroot_ref[...], axis=1, keepdims=True))
    score_ref[...] = s
    scaled_ref[...] = h_ref[...] * jnp.tanh(s)


def _score_kernel(aggr, h, wrel, brel, wroot):
    grid = (NP // N_BLK,)
    return pl.pallas_call(
        _score_body,
        grid=grid,
        in_specs=[
            pl.BlockSpec((N_BLK, DIM), lambda i: (i, 0)),
            pl.BlockSpec((N_BLK, DIM), lambda i: (i, 0)),
            pl.BlockSpec((1, DIM), lambda i: (0, 0)),
            pl.BlockSpec((1, 1), lambda i: (0, 0)),
            pl.BlockSpec((1, DIM), lambda i: (0, 0)),
        ],
        out_specs=[
            pl.BlockSpec((N_BLK, 1), lambda i: (i, 0)),
            pl.BlockSpec((N_BLK, DIM), lambda i: (i, 0)),
        ],
        out_shape=[
            jax.ShapeDtypeStruct((NP, 1), jnp.float32),
            jax.ShapeDtypeStruct((NP, DIM), jnp.float32),
        ],
    )(aggr, h, wrel.reshape(1, DIM), brel.reshape(1, 1), wroot.reshape(1, DIM))


def _rank_body(s_blk_ref, v_blk_ref, s_all_ref, v_all_ref, sel_ref, *, k):
    i0 = pl.program_id(0) * R_BLK
    s_i = jnp.where(v_blk_ref[...] > 0, s_blk_ref[...], _NEG)
    s_j = jnp.where(v_all_ref[...] > 0, s_all_ref[...], _NEG)
    j_idx = jax.lax.broadcasted_iota(jnp.int32, (1, NP), 1)
    i_idx = i0 + jax.lax.broadcasted_iota(jnp.int32, (R_BLK, 1), 0)
    beats = (s_j > s_i) | ((s_j == s_i) & (j_idx < i_idx))
    r = jnp.sum(beats.astype(jnp.float32), axis=1, keepdims=True)
    sel_ref[...] = jnp.where((v_blk_ref[...] > 0) & (r < k), 1.0, 0.0)


def _rank_kernel(score, valid, k):
    body = functools.partial(_rank_body, k=float(k))
    grid = (NP // R_BLK,)
    return pl.pallas_call(
        body,
        grid=grid,
        in_specs=[
            pl.BlockSpec((R_BLK, 1), lambda i: (i, 0)),
            pl.BlockSpec((R_BLK, 1), lambda i: (i, 0)),
            pl.BlockSpec((1, NP), lambda i: (0, 0)),
            pl.BlockSpec((1, NP), lambda i: (0, 0)),
        ],
        out_specs=pl.BlockSpec((R_BLK, 1), lambda i: (i, 0)),
        out_shape=jax.ShapeDtypeStruct((NP, 1), jnp.float32),
    )(score, valid, score.reshape(1, NP), valid.reshape(1, NP))


def _final_body(h1_ref, s1_ref, h2_ref, s2_ref, h3_ref, s3_ref,
                w1_ref, b1_ref, w2_ref, b2_ref, w3_ref, b3_ref, out_ref,
                *, k1, k2, k3):
    x1 = jnp.sum(h1_ref[...] * s1_ref[...], axis=0, keepdims=True) * (1.0 / k1)
    x2 = jnp.sum(h2_ref[...] * s2_ref[...], axis=0, keepdims=True) * (1.0 / k2)
    x3 = jnp.sum(h3_ref[...] * s3_ref[...], axis=0, keepdims=True) * (1.0 / k3)
    z = jnp.concatenate([x1, x2, x3], axis=1)
    z = jnp.maximum(
        jnp.dot(z, w1_ref[...], preferred_element_type=jnp.float32,
                precision=_HIGH) + b1_ref[...], 0.0)
    z = jnp.maximum(
        jnp.dot(z, w2_ref[...], preferred_element_type=jnp.float32,
                precision=_HIGH) + b2_ref[...], 0.0)
    out_ref[...] = (jnp.dot(z, w3_ref[...], preferred_element_type=jnp.float32,
                            precision=_HIGH) + b3_ref[...])


def _final_kernel(h1, s1, h2, s2, h3, s3, k1, k2, k3,
                  lin1_w, lin1_b, lin2_w, lin2_b, lin3_w, lin3_b):
    body = functools.partial(_final_body, k1=k1, k2=k2, k3=k3)
    return pl.pallas_call(
        body,
        out_shape=jax.ShapeDtypeStruct((1, 1), jnp.float32),
    )(h1, s1, h2, s2, h3, s3, lin1_w, lin1_b.reshape(1, DIM),
      lin2_w, lin2_b.reshape(1, DIM // 2), lin3_w, lin3_b.reshape(1, 1))


def kernel(x, edge_index, edge_attr, c1_w1, c1_b1, c1_w2, c1_b2, c1_root, c1_bias, c2_w1, c2_b1, c2_w2, c2_b2, c2_root, c2_bias, c3_w1, c3_b1, c3_w2, c3_b2, c3_root, c3_bias, p1_wrel, p1_brel, p1_wroot, p2_wrel, p2_brel, p2_wroot, p3_wrel, p3_brel, p3_wroot, lin1_w, lin1_b, lin2_w, lin2_b, lin3_w, lin3_b):
    src, dst = edge_index[0], edge_index[1]
    eap = _pad_rows(edge_attr, EP)

    conv = [(c1_w1, c1_b1, c1_w2, c1_b2, c1_root, c1_bias),
            (c2_w1, c2_b1, c2_w2, c2_b2, c2_root, c2_bias),
            (c3_w1, c3_b1, c3_w2, c3_b2, c3_root, c3_bias)]
    pool = [(p1_wrel, p1_brel, p1_wroot),
            (p2_wrel, p2_brel, p2_wroot),
            (p3_wrel, p3_brel, p3_wroot)]

    h_prev = _pad_rows(x, NP)
    valid = _pad_rows(jnp.ones((N_NODES, 1), jnp.float32), NP)
    eff_dst = dst
    mask = jnp.ones((N_EDGES,), dtype=bool)

    stages = []
    n = N_NODES
    for layer in range(3):
        w1, b1, w2, b2, root, bias = conv[layer]
        wrel, brel, wroot = pool[layer]

        xs = h_prev[src]
        msg = _msg_kernel(eap, _pad_rows(xs, EP), w1, b1, w2, b2)[:N_EDGES]
        aggr = jax.ops.segment_sum(msg, eff_dst, num_segments=NP + 1)[:NP]
        h = _combine_kernel(aggr, h_prev, root, bias)

        hs = h[src]
        aggr2 = jax.ops.segment_sum(hs, eff_dst, num_segments=NP + 1)[:NP]
        score, scaled = _score_kernel(aggr2, h, wrel, brel, wroot)

        k = -(-n // 2)
        sel = _rank_kernel(score, valid, k)
        mask = mask & (sel[src, 0] > 0) & (sel[dst, 0] > 0)
        eff_dst = jnp.where(mask, dst, NP)
        stages.append((scaled, sel, k))
        valid = sel
        h_prev = scaled
        n = k

    z = _final_kernel(stages[0][0], stages[0][1], stages[1][0], stages[1][1],
                      stages[2][0], stages[2][1],
                      stages[0][2], stages[1][2], stages[2][2],
                      lin1_w, lin1_b, lin2_w, lin2_b, lin3_w, lin3_b)
    return z.reshape(-1)


# speedup vs baseline: 1.8070x; 1.4750x over previous
import functools
import jax
import jax.numpy as jnp
from jax import lax
from jax.experimental import pallas as pl
from jax.experimental.pallas import tpu as pltpu
from jax.experimental.pallas import tpu_sc as plsc

N_NODES = 5000
N_EDGES = 10000
D_EDGE = 16
DIM = 64
HID = 128

E_BLK = 512
EP = 10240
NP = 5120
N_BLK = 256
R_BLK = 512

NC = 2
NS = 16
NW = NC * NS
E_W = EP // NW
NPA = NP + DIM
N_T = NPA // NS
N_W = NP // NS

_SC_MESH = plsc.VectorSubcoreMesh(core_axis_name="c", subcore_axis_name="s")
_SC_PARAMS = pltpu.CompilerParams(use_tc_tiling_on_sc=False,
                                  needs_layout_passes=False)


CH = 64
NCH = E_W // CH


@functools.partial(
    pl.kernel, mesh=_SC_MESH, compiler_params=_SC_PARAMS,
    out_type=jax.ShapeDtypeStruct((EP, DIM), jnp.float32),
    scratch_types=[
        pltpu.VMEM((NCH, CH), jnp.int32),
        pltpu.VMEM((E_W, DIM), jnp.float32),
        pltpu.SemaphoreType.DMA,
    ],
)
def _sc_gather(table_hbm, idx_hbm, out_hbm, idx_v, rows_v, sem):
    wid = lax.axis_index("s") * NC + lax.axis_index("c")
    base = wid * E_W
    pltpu.sync_copy(idx_hbm.at[wid], idx_v)
    copies = [
        pltpu.async_copy(table_hbm.at[idx_v.at[j]],
                         rows_v.at[pl.ds(j * CH, CH)], sem)
        for j in range(NCH)
    ]
    for c in copies:
        c.wait()
    pltpu.sync_copy(rows_v, out_hbm.at[pl.ds(base, E_W)])


@functools.partial(
    pl.kernel, mesh=_SC_MESH, compiler_params=_SC_PARAMS,
    out_type=jax.ShapeDtypeStruct((NC, NP, DIM), jnp.float32),
    scratch_types=[
        pltpu.VMEM((NCH, CH), jnp.int32),
        pltpu.VMEM((E_W, DIM), jnp.float32),
        pltpu.VMEM_SHARED((NPA, DIM), jnp.float32),
        pltpu.SemaphoreType.DMA,
    ],
)
def _sc_segsum_linear(rows_hbm, sidx_hbm, zeros_hbm, out_hbm,
                      sidx_v, rows_v, accum, sem):
    cid = lax.axis_index("c")
    sid = lax.axis_index("s")
    wid = sid * NC + cid
    base = wid * E_W
    pltpu.sync_copy(zeros_hbm.at[pl.ds(sid * N_T, N_T)],
                    accum.at[pl.ds(sid * N_T, N_T)])
    pltpu.sync_copy(sidx_hbm.at[wid], sidx_v)
    pltpu.sync_copy(rows_hbm.at[pl.ds(base, E_W)], rows_v)
    plsc.subcore_barrier()
    for j in range(NCH):
        pltpu.sync_copy(rows_v.at[pl.ds(j * CH, CH)],
                        accum.at[sidx_v.at[j]], add=True)
    plsc.subcore_barrier()
    pltpu.sync_copy(accum.at[pl.ds(sid * N_W, N_W)],
                    out_hbm.at[cid, pl.ds(sid * N_W, N_W)])


@functools.partial(
    pl.kernel, mesh=_SC_MESH, compiler_params=_SC_PARAMS,
    out_type=jax.ShapeDtypeStruct((NC, NP, DIM), jnp.float32),
    scratch_types=[
        pltpu.VMEM((NCH, CH), jnp.int32),
        pltpu.VMEM((NCH, CH), jnp.int32),
        pltpu.VMEM((E_W, DIM), jnp.float32),
        pltpu.VMEM_SHARED((NPA, DIM), jnp.float32),
        pltpu.SemaphoreType.DMA,
    ],
)
def _sc_segsum_gather(table_hbm, gidx_hbm, sidx_hbm, zeros_hbm, out_hbm,
                      gidx_v, sidx_v, rows_v, accum, sem):
    cid = lax.axis_index("c")
    sid = lax.axis_index("s")
    wid = sid * NC + cid
    pltpu.sync_copy(zeros_hbm.at[pl.ds(sid * N_T, N_T)],
                    accum.at[pl.ds(sid * N_T, N_T)])
    pltpu.sync_copy(gidx_hbm.at[wid], gidx_v)
    pltpu.sync_copy(sidx_hbm.at[wid], sidx_v)
    copies = [
        pltpu.async_copy(table_hbm.at[gidx_v.at[j]],
                         rows_v.at[pl.ds(j * CH, CH)], sem)
        for j in range(NCH)
    ]
    for c in copies:
        c.wait()
    plsc.subcore_barrier()
    for j in range(NCH):
        pltpu.sync_copy(rows_v.at[pl.ds(j * CH, CH)],
                        accum.at[sidx_v.at[j]], add=True)
    plsc.subcore_barrier()
    pltpu.sync_copy(accum.at[pl.ds(sid * N_W, N_W)],
                    out_hbm.at[cid, pl.ds(sid * N_W, N_W)])


@functools.partial(
    pl.kernel, mesh=_SC_MESH, compiler_params=_SC_PARAMS,
    out_type=jax.ShapeDtypeStruct((EP,), jnp.int32),
    scratch_types=[
        pltpu.VMEM((NP,), jnp.float32),
        pltpu.VMEM((E_W,), jnp.int32),
        pltpu.VMEM((E_W,), jnp.int32),
        pltpu.VMEM((E_W,), jnp.int32),
        pltpu.VMEM((E_W,), jnp.int32),
    ],
)
def _sc_remap(sel_hbm, src_hbm, dst_hbm, eff_hbm, out_hbm,
              sel_v, src_v, dst_v, eff_v, new_v):
    wid = lax.axis_index("s") * NC + lax.axis_index("c")
    base = wid * E_W
    pltpu.sync_copy(sel_hbm, sel_v)
    pltpu.sync_copy(src_hbm.at[pl.ds(base, E_W)], src_v)
    pltpu.sync_copy(dst_hbm.at[pl.ds(base, E_W)], dst_v)
    pltpu.sync_copy(eff_hbm.at[pl.ds(base, E_W)], eff_v)
    for j in range(E_W // 16):
        sl = pl.ds(j * 16, 16)
        sv = plsc.load_gather(sel_v, [src_v[sl]])
        dv = plsc.load_gather(sel_v, [dst_v[sl]])
        keep = (sv > 0.0) & (dv > 0.0) & (eff_v[sl] < NP)
        new_v[sl] = jnp.where(keep, dst_v[sl], NP)
    pltpu.sync_copy(new_v, out_hbm.at[pl.ds(base, E_W)])

_HIGH = jax.lax.Precision.HIGHEST
_NEG = -1.0e30


def _pad_rows(a, n):
    if a.shape[0] == n:
        return a
    return jnp.pad(a, ((0, n - a.shape[0]),) + ((0, 0),) * (a.ndim - 1))


def _x3dot(a, b):
    ah = a.astype(jnp.bfloat16)
    al = (a - ah.astype(jnp.float32)).astype(jnp.bfloat16)
    bh = b.astype(jnp.bfloat16)
    bl = (b - bh.astype(jnp.float32)).astype(jnp.bfloat16)
    d1 = jnp.dot(ah, bl, preferred_element_type=jnp.float32)
    d2 = jnp.dot(al, bh, preferred_element_type=jnp.float32)
    d3 = jnp.dot(ah, bh, preferred_element_type=jnp.float32)
    return (d1 + d2) + d3


def _msg_body(ea_ref, xs_ref, w1_ref, b1_ref, w2_ref, b2_ref, out_ref):
    h1 = jnp.maximum(_x3dot(ea_ref[...], w1_ref[...]) + b1_ref[...], 0.0)
    w = _x3dot(h1, w2_ref[...]) + b2_ref[...]
    p = w * jnp.repeat(xs_ref[...], DIM, axis=1)
    half = (DIM * DIM) // 2
    while half >= DIM:
        p = p[:, :half] + p[:, half:]
        half //= 2
    out_ref[...] = p


def _msg_kernel(ea, xs, w1, b1, w2, b2):
    grid = (EP // E_BLK,)
    return pl.pallas_call(
        _msg_body,
        grid=grid,
        in_specs=[
            pl.BlockSpec((E_BLK, D_EDGE), lambda i: (i, 0)),
            pl.BlockSpec((E_BLK, DIM), lambda i: (i, 0)),
            pl.BlockSpec((D_EDGE, HID), lambda i: (0, 0)),
            pl.BlockSpec((1, HID), lambda i: (0, 0)),
            pl.BlockSpec((HID, DIM * DIM), lambda i: (0, 0)),
            pl.BlockSpec((1, DIM * DIM), lambda i: (0, 0)),
        ],
        out_specs=pl.BlockSpec((E_BLK, DIM), lambda i: (i, 0)),
        out_shape=jax.ShapeDtypeStruct((EP, DIM), jnp.float32),
    )(ea, xs, w1, b1.reshape(1, HID), w2, b2.reshape(1, DIM * DIM))


def _combine_body(a0_ref, a1_ref, x_ref, root_ref, bias_ref, out_ref):
    out_ref[...] = jnp.maximum(
        a0_ref[...] + a1_ref[...]
        + jnp.dot(x_ref[...], root_ref[...], preferred_element_type=jnp.float32,
                  precision=_HIGH)
        + bias_ref[...], 0.0)


def _combine_kernel(a0, a1, x, root, bias):
    grid = (NP // N_BLK,)
    return pl.pallas_call(
        _combine_body,
        grid=grid,
        in_specs=[
            pl.BlockSpec((N_BLK, DIM), lambda i: (i, 0)),
            pl.BlockSpec((N_BLK, DIM), lambda i: (i, 0)),
            pl.BlockSpec((N_BLK, DIM), lambda i: (i, 0)),
            pl.BlockSpec((DIM, DIM), lambda i: (0, 0)),
            pl.BlockSpec((1, DIM), lambda i: (0, 0)),
        ],
        out_specs=pl.BlockSpec((N_BLK, DIM), lambda i: (i, 0)),
        out_shape=jax.ShapeDtypeStruct((NP, DIM), jnp.float32),
    )(a0, a1, x, root, bias.reshape(1, DIM))


def _score_body(a0_ref, a1_ref, h_ref, wrel_ref, brel_ref, wroot_ref,
                score_ref, scaled_ref):
    aggr = a0_ref[...] + a1_ref[...]
    s = (jnp.sum(aggr * wrel_ref[...], axis=1, keepdims=True)
         + brel_ref[...]
         + jnp.sum(h_ref[...] * wroot_ref[...], axis=1, keepdims=True))
    score_ref[...] = s
    scaled_ref[...] = h_ref[...] * jnp.tanh(s)


def _score_kernel(a0, a1, h, wrel, brel, wroot):
    grid = (NP // N_BLK,)
    return pl.pallas_call(
        _score_body,
        grid=grid,
        in_specs=[
            pl.BlockSpec((N_BLK, DIM), lambda i: (i, 0)),
            pl.BlockSpec((N_BLK, DIM), lambda i: (i, 0)),
            pl.BlockSpec((N_BLK, DIM), lambda i: (i, 0)),
            pl.BlockSpec((1, DIM), lambda i: (0, 0)),
            pl.BlockSpec((1, 1), lambda i: (0, 0)),
            pl.BlockSpec((1, DIM), lambda i: (0, 0)),
        ],
        out_specs=[
            pl.BlockSpec((N_BLK, 1), lambda i: (i, 0)),
            pl.BlockSpec((N_BLK, DIM), lambda i: (i, 0)),
        ],
        out_shape=[
            jax.ShapeDtypeStruct((NP, 1), jnp.float32),
            jax.ShapeDtypeStruct((NP, DIM), jnp.float32),
        ],
    )(a0, a1, h, wrel.reshape(1, DIM), brel.reshape(1, 1),
      wroot.reshape(1, DIM))


def _rank_body(s_blk_ref, v_blk_ref, o_blk_ref, s_all_ref, v_all_ref,
               o_all_ref, sel_ref, ord_ref, *, k):
    s_i = jnp.where(v_blk_ref[...] > 0, s_blk_ref[...], _NEG)
    s_j = jnp.where(v_all_ref[...] > 0, s_all_ref[...], _NEG)
    o_i = o_blk_ref[...]
    o_j = o_all_ref[...]
    beats = (s_j > s_i) | ((s_j == s_i) & (o_j < o_i))
    r = jnp.sum(beats.astype(jnp.float32), axis=1, keepdims=True)
    sel_ref[...] = jnp.where((v_blk_ref[...] > 0) & (r < k), 1.0, 0.0)
    ord_ref[...] = r


def _rank_kernel(score, valid, ordr, k):
    body = functools.partial(_rank_body, k=float(k))
    grid = (NP // R_BLK,)
    return pl.pallas_call(
        body,
        grid=grid,
        in_specs=[
            pl.BlockSpec((R_BLK, 1), lambda i: (i, 0)),
            pl.BlockSpec((R_BLK, 1), lambda i: (i, 0)),
            pl.BlockSpec((R_BLK, 1), lambda i: (i, 0)),
            pl.BlockSpec((1, NP), lambda i: (0, 0)),
            pl.BlockSpec((1, NP), lambda i: (0, 0)),
            pl.BlockSpec((1, NP), lambda i: (0, 0)),
        ],
        out_specs=[
            pl.BlockSpec((R_BLK, 1), lambda i: (i, 0)),
            pl.BlockSpec((R_BLK, 1), lambda i: (i, 0)),
        ],
        out_shape=[
            jax.ShapeDtypeStruct((NP, 1), jnp.float32),
            jax.ShapeDtypeStruct((NP, 1), jnp.float32),
        ],
    )(score, valid, ordr, score.reshape(1, NP), valid.reshape(1, NP),
      ordr.reshape(1, NP))


def _final_body(h1_ref, s1_ref, h2_ref, s2_ref, h3_ref, s3_ref,
                w1_ref, b1_ref, w2_ref, b2_ref, w3_ref, b3_ref, out_ref,
                *, k1, k2, k3):
    x1 = jnp.sum(h1_ref[...] * s1_ref[...], axis=0, keepdims=True) * (1.0 / k1)
    x2 = jnp.sum(h2_ref[...] * s2_ref[...], axis=0, keepdims=True) * (1.0 / k2)
    x3 = jnp.sum(h3_ref[...] * s3_ref[...], axis=0, keepdims=True) * (1.0 / k3)
    z = jnp.concatenate([x1, x2, x3], axis=1)
    z = jnp.maximum(
        jnp.dot(z, w1_ref[...], preferred_element_type=jnp.float32,
                precision=_HIGH) + b1_ref[...], 0.0)
    z = jnp.maximum(
        jnp.dot(z, w2_ref[...], preferred_element_type=jnp.float32,
                precision=_HIGH) + b2_ref[...], 0.0)
    out_ref[...] = (jnp.dot(z, w3_ref[...], preferred_element_type=jnp.float32,
                            precision=_HIGH) + b3_ref[...])


def _final_kernel(h1, s1, h2, s2, h3, s3, k1, k2, k3,
                  lin1_w, lin1_b, lin2_w, lin2_b, lin3_w, lin3_b):
    body = functools.partial(_final_body, k1=k1, k2=k2, k3=k3)
    return pl.pallas_call(
        body,
        out_shape=jax.ShapeDtypeStruct((1, 1), jnp.float32),
    )(h1, s1, h2, s2, h3, s3, lin1_w, lin1_b.reshape(1, DIM),
      lin2_w, lin2_b.reshape(1, DIM // 2), lin3_w, lin3_b.reshape(1, 1))


def kernel(x, edge_index, edge_attr, c1_w1, c1_b1, c1_w2, c1_b2, c1_root, c1_bias, c2_w1, c2_b1, c2_w2, c2_b2, c2_root, c2_bias, c3_w1, c3_b1, c3_w2, c3_b2, c3_root, c3_bias, p1_wrel, p1_brel, p1_wroot, p2_wrel, p2_brel, p2_wroot, p3_wrel, p3_brel, p3_wroot, lin1_w, lin1_b, lin2_w, lin2_b, lin3_w, lin3_b):
    src, dst = edge_index[0], edge_index[1]
    eap = _pad_rows(edge_attr, EP)

    conv = [(c1_w1, c1_b1, c1_w2, c1_b2, c1_root, c1_bias),
            (c2_w1, c2_b1, c2_w2, c2_b2, c2_root, c2_bias),
            (c3_w1, c3_b1, c3_w2, c3_b2, c3_root, c3_bias)]
    pool = [(p1_wrel, p1_brel, p1_wroot),
            (p2_wrel, p2_brel, p2_wroot),
            (p3_wrel, p3_brel, p3_wroot)]

    h_prev = _pad_rows(x, NP)
    valid = _pad_rows(jnp.ones((N_NODES, 1), jnp.float32), NP)
    ordr = jnp.arange(NP, dtype=jnp.float32).reshape(NP, 1)
    srcp = jnp.pad(src, (0, EP - N_EDGES))
    dstp = jnp.pad(dst, (0, EP - N_EDGES))
    eff_dst = jnp.pad(dst, (0, EP - N_EDGES), constant_values=NP)
    zeros = jnp.zeros((NPA, DIM), jnp.float32)

    stages = []
    n = N_NODES
    for layer in range(3):
        w1, b1, w2, b2, root, bias = conv[layer]
        wrel, brel, wroot = pool[layer]

        src3 = srcp.reshape(NW, NCH, CH)
        eff3 = eff_dst.reshape(NW, NCH, CH)
        xs = _sc_gather(h_prev, src3)
        msg = _msg_kernel(eap, xs, w1, b1, w2, b2)
        ap = _sc_segsum_linear(msg, eff3, zeros)
        h = jax.nn.relu((ap[0] + ap[1]) + h_prev @ root + bias)

        a2p = _sc_segsum_gather(h, src3, eff3, zeros)
        score = (a2p[0] + a2p[1]) @ wrel + brel + h @ wroot
        scaled = h * jnp.tanh(score)

        k = -(-n // 2)
        sel, ordr = _rank_kernel(score, valid, ordr, k)
        if layer < 2:
            eff_dst = _sc_remap(sel.reshape(NP), srcp, dstp, eff_dst)
        stages.append((scaled, sel, k))
        valid = sel
        h_prev = scaled
        n = k

    zs = []
    for scl, sl, kk in stages:
        zs.append(jnp.sum(scl * sl, axis=0, keepdims=True) / kk)
    z = jnp.concatenate(zs, axis=1)
    z = jax.nn.relu(z @ lin1_w + lin1_b)
    z = jax.nn.relu(z @ lin2_w + lin2_b)
    z = z @ lin3_w + lin3_b
    return z.reshape(-1)
